# baseline (speedup 1.0000x reference)
import functools

import jax
import jax.numpy as jnp
from jax import lax
from jax.experimental import pallas as pl
from jax.experimental.pallas import tpu as pltpu
from jax.experimental.pallas import tpu_sc as plsc


def _sc_gather(emb, xflat, *, chunk=128):
    n_total, emb_dim = xflat.shape[0], emb.shape[1]
    info = plsc.get_sparse_core_info()
    nc, ns = info.num_cores, info.num_subcores
    nw = nc * ns
    n_per_w = n_total // nw
    assert n_per_w * nw == n_total and n_per_w % chunk == 0
    n_iters = n_per_w // chunk

    mesh = plsc.VectorSubcoreMesh(core_axis_name="c", subcore_axis_name="s")

    assert n_iters % 2 == 0

    @functools.partial(
        pl.kernel,
        mesh=mesh,
        out_type=jax.ShapeDtypeStruct((n_total, emb_dim), jnp.float32),
        scratch_types=[
            pltpu.VMEM((chunk,), jnp.int32),
            pltpu.VMEM((chunk,), jnp.int32),
            pltpu.VMEM((chunk, emb_dim), jnp.float32),
            pltpu.VMEM((chunk, emb_dim), jnp.float32),
            pltpu.SemaphoreType.DMA,
            pltpu.SemaphoreType.DMA,
        ],
    )
    def gather_kernel(emb_hbm, idx_hbm, out_hbm, idx_a, idx_b, rows_a,
                      rows_b, sem_g, sem_o):
        wid = lax.axis_index("s") * nc + lax.axis_index("c")
        base = wid * n_per_w

        def phase(i, idx_cur, rows_cur, idx_nxt, rows_nxt):
            off = base + i * chunk
            pltpu.make_async_copy(emb_hbm.at[idx_cur], rows_cur, sem_g).wait()

            @pl.when(i > 0)
            def _():
                pltpu.make_async_copy(
                    rows_nxt, out_hbm.at[pl.ds(off, chunk)], sem_o).wait()

            pltpu.async_copy(rows_cur, out_hbm.at[pl.ds(off, chunk)], sem_o)

            @pl.when(i + 1 < n_iters)
            def _():
                pltpu.sync_copy(
                    idx_hbm.at[pl.ds(off + chunk, chunk)], idx_nxt)
                pltpu.async_copy(emb_hbm.at[idx_nxt], rows_nxt, sem_g)

        pltpu.sync_copy(idx_hbm.at[pl.ds(base, chunk)], idx_a)
        pltpu.async_copy(emb_hbm.at[idx_a], rows_a, sem_g)

        def body(j, carry):
            phase(2 * j, idx_a, rows_a, idx_b, rows_b)
            phase(2 * j + 1, idx_b, rows_b, idx_a, rows_a)
            return carry

        lax.fori_loop(0, n_iters // 2, body, 0)
        pltpu.make_async_copy(
            rows_b, out_hbm.at[pl.ds(base, chunk)], sem_o).wait()

    return gather_kernel(emb, xflat)


def _fc_in_kernel(g_ref, w_ref, b_ref, o_ref, *, c_tile):
    k = pl.program_id(0)

    @pl.when(k == 0)
    def _():
        o_ref[...] = jnp.broadcast_to(b_ref[...], o_ref.shape)

    acc = o_ref[...]
    for c in range(c_tile):
        acc += lax.dot_general(
            g_ref[:, c, :], w_ref[:, c, :], (((1,), (1,)), ((), ())),
            preferred_element_type=jnp.float32)
    o_ref[...] = acc


def _fc_in(g, w_in, b_in, *, c_tile=8):
    b, ctx, emb_dim = g.shape
    v = w_in.shape[0]
    w3 = w_in.reshape(v, ctx, emb_dim)
    n_c = ctx // c_tile
    assert n_c * c_tile == ctx
    return pl.pallas_call(
        functools.partial(_fc_in_kernel, c_tile=c_tile),
        grid=(n_c,),
        in_specs=[
            pl.BlockSpec((b, c_tile, emb_dim), lambda k: (0, k, 0)),
            pl.BlockSpec((v, c_tile, emb_dim), lambda k: (0, k, 0)),
            pl.BlockSpec((1, v), lambda k: (0, 0)),
        ],
        out_specs=pl.BlockSpec((b, v), lambda k: (0, 0)),
        out_shape=jax.ShapeDtypeStruct((b, v), jnp.float32),
    )(g, w3, b_in)


def _fc_out_kernel(h_ref, w_ref, b_ref, o_ref):
    o_ref[...] = lax.dot_general(
        h_ref[...], w_ref[...], (((1,), (1,)), ((), ())),
        preferred_element_type=jnp.float32) + b_ref[...]


def _fc_out_part_kernel(h_ref, w_ref, b_ref, prev_ref, o_ref):
    del prev_ref
    _fc_out_kernel(h_ref, w_ref, b_ref, o_ref)


def _fc_out_part(h, w_out, b_out, prev, *, part, batch_total, v_tile=4096):
    bh, v = h.shape
    vocab = w_out.shape[0]
    n_v = pl.cdiv(vocab, v_tile)
    in_specs = [
        pl.BlockSpec((bh, v), lambda j: (0, 0)),
        pl.BlockSpec((v_tile, v), lambda j: (j, 0)),
        pl.BlockSpec((1, v_tile), lambda j: (0, j)),
    ]
    out_spec = pl.BlockSpec((bh, v_tile), lambda j, part=part: (part, j))
    out_shape = jax.ShapeDtypeStruct((batch_total, vocab), jnp.float32)
    if prev is None:
        return pl.pallas_call(
            _fc_out_kernel,
            grid=(n_v,),
            in_specs=in_specs,
            out_specs=out_spec,
            out_shape=out_shape,
        )(h, w_out, b_out)
    return pl.pallas_call(
        _fc_out_part_kernel,
        grid=(n_v,),
        in_specs=in_specs + [pl.BlockSpec(memory_space=pl.ANY)],
        out_specs=out_spec,
        out_shape=out_shape,
        input_output_aliases={3: 0},
    )(h, w_out, b_out, prev)


def kernel(x, emb, W_in, b_in, W_out, b_out):
    b, ctx = x.shape
    emb_dim = emb.shape[1]
    nparts = 2
    bh = b // nparts
    assert bh * nparts == b
    b_in2 = b_in.reshape(1, -1)
    b_out2 = b_out.reshape(1, -1)
    hs = []
    for i in range(nparts):
        xi = x[i * bh:(i + 1) * bh].reshape(-1)
        gi = _sc_gather(emb, xi, chunk=80)
        gi = gi.reshape(bh, ctx, emb_dim)
        hs.append(_fc_in(gi, W_in, b_in2))
    out = None
    for i, hi in enumerate(hs):
        out = _fc_out_part(hi, W_out, b_out2, out, part=i, batch_total=b)
    return out


# speedup vs baseline: 1.0799x; 1.0799x over previous
import functools

import jax
import jax.numpy as jnp
from jax import lax
from jax.experimental import pallas as pl
from jax.experimental.pallas import tpu as pltpu
from jax.experimental.pallas import tpu_sc as plsc


def _sc_gather(emb, xflat, *, chunk=64):
    n_total, emb_dim = xflat.shape[0], emb.shape[1]
    info = plsc.get_sparse_core_info()
    nc, ns = info.num_cores, info.num_subcores
    nw = nc * ns
    n_per_w = n_total // nw
    assert n_per_w * nw == n_total and n_per_w % chunk == 0
    n_iters = n_per_w // chunk
    assert n_iters % 4 == 0 and n_iters >= 8

    mesh = plsc.VectorSubcoreMesh(core_axis_name="c", subcore_axis_name="s")

    @functools.partial(
        pl.kernel,
        mesh=mesh,
        out_type=jax.ShapeDtypeStruct((n_total, emb_dim), jnp.float32),
        scratch_types=[
            pltpu.VMEM((4, chunk), jnp.int32),
            pltpu.VMEM((chunk, emb_dim), jnp.float32),
            pltpu.VMEM((chunk, emb_dim), jnp.float32),
            pltpu.VMEM((chunk, emb_dim), jnp.float32),
            pltpu.VMEM((chunk, emb_dim), jnp.float32),
            pltpu.SemaphoreType.DMA,
            pltpu.SemaphoreType.DMA,
        ],
    )
    def gather_kernel(emb_hbm, idx_hbm, out_hbm, idx_v, r0, r1, r2, r3,
                      sem_g, sem_o):
        rows = [r0, r1, r2, r3]
        wid = lax.axis_index("s") * nc + lax.axis_index("c")
        base = wid * n_per_w

        def phase(i, p, guard_lo, guard_hi):
            off = base + i * chunk
            buf = rows[p]
            pltpu.make_async_copy(
                emb_hbm.at[idx_v.at[p]], buf, sem_g).wait()

            def wait_out():
                pltpu.make_async_copy(
                    rows[(p + 2) % 4], out_hbm.at[pl.ds(off, chunk)],
                    sem_o).wait()
            if guard_lo is None:
                wait_out()
            else:
                pl.when(guard_lo)(wait_out)

            pltpu.async_copy(buf, out_hbm.at[pl.ds(off, chunk)], sem_o)

            def fire_next():
                nxt = (p + 2) % 4
                pltpu.sync_copy(
                    idx_hbm.at[pl.ds(off + 2 * chunk, chunk)], idx_v.at[nxt])
                pltpu.async_copy(
                    emb_hbm.at[idx_v.at[nxt]], rows[nxt], sem_g)
            if guard_hi is None:
                fire_next()
            else:
                pl.when(guard_hi)(fire_next)

        pltpu.sync_copy(idx_hbm.at[pl.ds(base, chunk)], idx_v.at[0])
        pltpu.async_copy(emb_hbm.at[idx_v.at[0]], rows[0], sem_g)
        pltpu.sync_copy(idx_hbm.at[pl.ds(base + chunk, chunk)], idx_v.at[1])
        pltpu.async_copy(emb_hbm.at[idx_v.at[1]], rows[1], sem_g)

        n_j = n_iters // 4

        def body(j, carry):
            phase(4 * j + 0, 0, j > 0, None)
            phase(4 * j + 1, 1, j > 0, None)
            phase(4 * j + 2, 2, None, j < n_j - 1)
            phase(4 * j + 3, 3, None, j < n_j - 1)
            return carry

        lax.fori_loop(0, n_j, body, 0)
        pltpu.make_async_copy(
            rows[2], out_hbm.at[pl.ds(base, chunk)], sem_o).wait()
        pltpu.make_async_copy(
            rows[3], out_hbm.at[pl.ds(base, chunk)], sem_o).wait()

    return gather_kernel(emb, xflat)


def _fc_in_kernel(g_ref, w_ref, b_ref, o_ref, *, c_tile):
    k = pl.program_id(0)

    @pl.when(k == 0)
    def _():
        o_ref[...] = jnp.broadcast_to(b_ref[...], o_ref.shape)

    acc = o_ref[...]
    for c in range(c_tile):
        acc += lax.dot_general(
            g_ref[:, c, :], w_ref[:, c, :], (((1,), (1,)), ((), ())),
            preferred_element_type=jnp.float32)
    o_ref[...] = acc


def _fc_in(g, w_in, b_in, *, c_tile=8):
    b, ctx, emb_dim = g.shape
    v = w_in.shape[0]
    w3 = w_in.reshape(v, ctx, emb_dim)
    n_c = ctx // c_tile
    assert n_c * c_tile == ctx
    return pl.pallas_call(
        functools.partial(_fc_in_kernel, c_tile=c_tile),
        grid=(n_c,),
        in_specs=[
            pl.BlockSpec((b, c_tile, emb_dim), lambda k: (0, k, 0)),
            pl.BlockSpec((v, c_tile, emb_dim), lambda k: (0, k, 0)),
            pl.BlockSpec((1, v), lambda k: (0, 0)),
        ],
        out_specs=pl.BlockSpec((b, v), lambda k: (0, 0)),
        out_shape=jax.ShapeDtypeStruct((b, v), jnp.float32),
    )(g, w3, b_in)


def _fc_out_kernel(h_ref, w_ref, b_ref, o_ref):
    o_ref[...] = lax.dot_general(
        h_ref[...], w_ref[...], (((1,), (1,)), ((), ())),
        preferred_element_type=jnp.float32) + b_ref[...]


def _fc_out(h, w_out, b_out, *, v_tile=4096):
    b, v = h.shape
    vocab = w_out.shape[0]
    n_v = pl.cdiv(vocab, v_tile)
    return pl.pallas_call(
        _fc_out_kernel,
        grid=(n_v,),
        in_specs=[
            pl.BlockSpec((b, v), lambda j: (0, 0)),
            pl.BlockSpec((v_tile, v), lambda j: (j, 0)),
            pl.BlockSpec((1, v_tile), lambda j: (0, j)),
        ],
        out_specs=pl.BlockSpec((b, v_tile), lambda j: (0, j)),
        out_shape=jax.ShapeDtypeStruct((b, vocab), jnp.float32),
    )(h, w_out, b_out)


def kernel(x, emb, W_in, b_in, W_out, b_out):
    b, ctx = x.shape
    emb_dim = emb.shape[1]
    xflat = x.reshape(-1)
    g = _sc_gather(emb, xflat)
    g = g.reshape(b, ctx, emb_dim)
    h = _fc_in(g, W_in, b_in.reshape(1, -1))
    return _fc_out(h, W_out, b_out.reshape(1, -1))


# speedup vs baseline: 1.0974x; 1.0162x over previous
import functools

import jax
import jax.numpy as jnp
from jax import lax
from jax.experimental import pallas as pl
from jax.experimental.pallas import tpu as pltpu
from jax.experimental.pallas import tpu_sc as plsc


def _sc_gather(emb, xflat, *, chunk=80):
    n_total, emb_dim = xflat.shape[0], emb.shape[1]
    info = plsc.get_sparse_core_info()
    nc, ns = info.num_cores, info.num_subcores
    nw = nc * ns
    n_per_w = n_total // nw
    assert n_per_w * nw == n_total and n_per_w % chunk == 0
    n_iters = n_per_w // chunk
    assert n_iters % 4 == 0 and n_iters >= 8

    mesh = plsc.VectorSubcoreMesh(core_axis_name="c", subcore_axis_name="s")

    @functools.partial(
        pl.kernel,
        mesh=mesh,
        out_type=jax.ShapeDtypeStruct((n_total, emb_dim), jnp.float32),
        scratch_types=[
            pltpu.VMEM((4, chunk), jnp.int32),
            pltpu.VMEM((chunk, emb_dim), jnp.float32),
            pltpu.VMEM((chunk, emb_dim), jnp.float32),
            pltpu.VMEM((chunk, emb_dim), jnp.float32),
            pltpu.VMEM((chunk, emb_dim), jnp.float32),
            pltpu.SemaphoreType.DMA,
            pltpu.SemaphoreType.DMA,
        ],
    )
    def gather_kernel(emb_hbm, idx_hbm, out_hbm, idx_v, r0, r1, r2, r3,
                      sem_g, sem_o):
        rows = [r0, r1, r2, r3]
        wid = lax.axis_index("s") * nc + lax.axis_index("c")
        base = wid * n_per_w

        def phase(i, p, guard_lo, guard_hi):
            off = base + i * chunk
            buf = rows[p]
            pltpu.make_async_copy(
                emb_hbm.at[idx_v.at[p]], buf, sem_g).wait()

            def wait_out():
                pltpu.make_async_copy(
                    rows[(p + 2) % 4], out_hbm.at[pl.ds(off, chunk)],
                    sem_o).wait()
            if guard_lo is None:
                wait_out()
            else:
                pl.when(guard_lo)(wait_out)

            pltpu.async_copy(buf, out_hbm.at[pl.ds(off, chunk)], sem_o)

            def fire_next():
                nxt = (p + 2) % 4
                pltpu.sync_copy(
                    idx_hbm.at[pl.ds(off + 2 * chunk, chunk)], idx_v.at[nxt])
                pltpu.async_copy(
                    emb_hbm.at[idx_v.at[nxt]], rows[nxt], sem_g)
            if guard_hi is None:
                fire_next()
            else:
                pl.when(guard_hi)(fire_next)

        pltpu.sync_copy(idx_hbm.at[pl.ds(base, chunk)], idx_v.at[0])
        pltpu.async_copy(emb_hbm.at[idx_v.at[0]], rows[0], sem_g)
        pltpu.sync_copy(idx_hbm.at[pl.ds(base + chunk, chunk)], idx_v.at[1])
        pltpu.async_copy(emb_hbm.at[idx_v.at[1]], rows[1], sem_g)

        n_j = n_iters // 4

        def body(j, carry):
            phase(4 * j + 0, 0, j > 0, None)
            phase(4 * j + 1, 1, j > 0, None)
            phase(4 * j + 2, 2, None, j < n_j - 1)
            phase(4 * j + 3, 3, None, j < n_j - 1)
            return carry

        lax.fori_loop(0, n_j, body, 0)
        pltpu.make_async_copy(
            rows[2], out_hbm.at[pl.ds(base, chunk)], sem_o).wait()
        pltpu.make_async_copy(
            rows[3], out_hbm.at[pl.ds(base, chunk)], sem_o).wait()

    return gather_kernel(emb, xflat)


def _fc_in_kernel(g_ref, w_ref, b_ref, o_ref, *, c_tile):
    k = pl.program_id(0)

    @pl.when(k == 0)
    def _():
        o_ref[...] = jnp.broadcast_to(b_ref[...], o_ref.shape)

    acc = o_ref[...]
    for c in range(c_tile):
        acc += lax.dot_general(
            g_ref[:, c, :], w_ref[:, c, :], (((1,), (1,)), ((), ())),
            preferred_element_type=jnp.float32)
    o_ref[...] = acc


def _fc_in(g, w_in, b_in, *, c_tile=8):
    b, ctx, emb_dim = g.shape
    v = w_in.shape[0]
    w3 = w_in.reshape(v, ctx, emb_dim)
    n_c = ctx // c_tile
    assert n_c * c_tile == ctx
    return pl.pallas_call(
        functools.partial(_fc_in_kernel, c_tile=c_tile),
        grid=(n_c,),
        in_specs=[
            pl.BlockSpec((b, c_tile, emb_dim), lambda k: (0, k, 0)),
            pl.BlockSpec((v, c_tile, emb_dim), lambda k: (0, k, 0)),
            pl.BlockSpec((1, v), lambda k: (0, 0)),
        ],
        out_specs=pl.BlockSpec((b, v), lambda k: (0, 0)),
        out_shape=jax.ShapeDtypeStruct((b, v), jnp.float32),
    )(g, w3, b_in)


def _fc_out_kernel(h_ref, w_ref, b_ref, o_ref):
    o_ref[...] = lax.dot_general(
        h_ref[...], w_ref[...], (((1,), (1,)), ((), ())),
        preferred_element_type=jnp.float32) + b_ref[...]


def _fc_out(h, w_out, b_out, *, v_tile=4096):
    b, v = h.shape
    vocab = w_out.shape[0]
    n_v = pl.cdiv(vocab, v_tile)
    return pl.pallas_call(
        _fc_out_kernel,
        grid=(n_v,),
        in_specs=[
            pl.BlockSpec((b, v), lambda j: (0, 0)),
            pl.BlockSpec((v_tile, v), lambda j: (j, 0)),
            pl.BlockSpec((1, v_tile), lambda j: (0, j)),
        ],
        out_specs=pl.BlockSpec((b, v_tile), lambda j: (0, j)),
        out_shape=jax.ShapeDtypeStruct((b, vocab), jnp.float32),
    )(h, w_out, b_out)


def kernel(x, emb, W_in, b_in, W_out, b_out):
    b, ctx = x.shape
    emb_dim = emb.shape[1]
    xflat = x.reshape(-1)
    g = _sc_gather(emb, xflat)
    g = g.reshape(b, ctx, emb_dim)
    h = _fc_in(g, W_in, b_in.reshape(1, -1))
    return _fc_out(h, W_out, b_out.reshape(1, -1))


# speedup vs baseline: 1.0991x; 1.0015x over previous
import functools

import jax
import jax.numpy as jnp
from jax import lax
from jax.experimental import pallas as pl
from jax.experimental.pallas import tpu as pltpu
from jax.experimental.pallas import tpu_sc as plsc


def _sc_gather(emb, xflat, *, chunk=80):
    n_total, emb_dim = xflat.shape[0], emb.shape[1]
    info = plsc.get_sparse_core_info()
    nc, ns = info.num_cores, info.num_subcores
    nw = nc * ns
    n_per_w = n_total // nw
    assert n_per_w * nw == n_total and n_per_w % chunk == 0
    n_iters = n_per_w // chunk
    ring = 8
    ahead = 4
    assert n_iters % ring == 0 and n_iters >= 2 * ring

    mesh = plsc.VectorSubcoreMesh(core_axis_name="c", subcore_axis_name="s")

    @functools.partial(
        pl.kernel,
        mesh=mesh,
        out_type=jax.ShapeDtypeStruct((n_total, emb_dim), jnp.float32),
        scratch_types=[
            pltpu.VMEM((ring, chunk), jnp.int32),
        ] + [pltpu.VMEM((chunk, emb_dim), jnp.float32)] * ring + [
            pltpu.SemaphoreType.DMA,
            pltpu.SemaphoreType.DMA,
        ],
    )
    def gather_kernel(emb_hbm, idx_hbm, out_hbm, idx_v, *bufs_and_sems):
        rows = list(bufs_and_sems[:ring])
        sem_g, sem_o = bufs_and_sems[ring:]
        wid = lax.axis_index("s") * nc + lax.axis_index("c")
        base = wid * n_per_w

        def phase(i, p, j, n_j):
            off = base + i * chunk
            buf = rows[p]
            pltpu.make_async_copy(
                emb_hbm.at[idx_v.at[p]], buf, sem_g).wait()

            def wait_out():
                pltpu.make_async_copy(
                    rows[(p + ahead) % ring], out_hbm.at[pl.ds(off, chunk)],
                    sem_o).wait()
            if p >= ahead:
                wait_out()
            else:
                pl.when(j > 0)(wait_out)

            pltpu.async_copy(buf, out_hbm.at[pl.ds(off, chunk)], sem_o)

            @pl.when(i + ahead < n_iters)
            def _():
                nxt = (p + ahead) % ring
                pltpu.sync_copy(
                    idx_hbm.at[pl.ds(off + ahead * chunk, chunk)],
                    idx_v.at[nxt])
                pltpu.async_copy(
                    emb_hbm.at[idx_v.at[nxt]], rows[nxt], sem_g)

        for p in range(ahead):
            pltpu.sync_copy(
                idx_hbm.at[pl.ds(base + p * chunk, chunk)], idx_v.at[p])
            pltpu.async_copy(emb_hbm.at[idx_v.at[p]], rows[p], sem_g)

        n_j = n_iters // ring

        def body(j, carry):
            for p in range(ring):
                phase(ring * j + p, p, j, n_j)
            return carry

        lax.fori_loop(0, n_j, body, 0)
        for p in range(ahead):
            pltpu.make_async_copy(
                rows[(ring - ahead + p) % ring],
                out_hbm.at[pl.ds(base, chunk)], sem_o).wait()

    return gather_kernel(emb, xflat)


def _fc_in_kernel(g_ref, w_ref, b_ref, o_ref, *, c_tile):
    k = pl.program_id(0)

    @pl.when(k == 0)
    def _():
        o_ref[...] = jnp.broadcast_to(b_ref[...], o_ref.shape)

    acc = o_ref[...]
    for c in range(c_tile):
        acc += lax.dot_general(
            g_ref[:, c, :], w_ref[:, c, :], (((1,), (1,)), ((), ())),
            preferred_element_type=jnp.float32)
    o_ref[...] = acc


def _fc_in(g, w_in, b_in, *, c_tile=40):
    b, ctx, emb_dim = g.shape
    v = w_in.shape[0]
    w3 = w_in.reshape(v, ctx, emb_dim)
    n_c = ctx // c_tile
    assert n_c * c_tile == ctx
    return pl.pallas_call(
        functools.partial(_fc_in_kernel, c_tile=c_tile),
        grid=(n_c,),
        in_specs=[
            pl.BlockSpec((b, c_tile, emb_dim), lambda k: (0, k, 0)),
            pl.BlockSpec((v, c_tile, emb_dim), lambda k: (0, k, 0)),
            pl.BlockSpec((1, v), lambda k: (0, 0)),
        ],
        out_specs=pl.BlockSpec((b, v), lambda k: (0, 0)),
        out_shape=jax.ShapeDtypeStruct((b, v), jnp.float32),
    )(g, w3, b_in)


def _fc_out_kernel(h_ref, w_ref, b_ref, o_ref):
    o_ref[...] = lax.dot_general(
        h_ref[...], w_ref[...], (((1,), (1,)), ((), ())),
        preferred_element_type=jnp.float32) + b_ref[...]


def _fc_out(h, w_out, b_out, *, v_tile=4096):
    b, v = h.shape
    vocab = w_out.shape[0]
    n_v = pl.cdiv(vocab, v_tile)
    return pl.pallas_call(
        _fc_out_kernel,
        grid=(n_v,),
        in_specs=[
            pl.BlockSpec((b, v), lambda j: (0, 0)),
            pl.BlockSpec((v_tile, v), lambda j: (j, 0)),
            pl.BlockSpec((1, v_tile), lambda j: (0, j)),
        ],
        out_specs=pl.BlockSpec((b, v_tile), lambda j: (0, j)),
        out_shape=jax.ShapeDtypeStruct((b, vocab), jnp.float32),
    )(h, w_out, b_out)


def kernel(x, emb, W_in, b_in, W_out, b_out):
    b, ctx = x.shape
    emb_dim = emb.shape[1]
    xflat = x.reshape(-1)
    g = _sc_gather(emb, xflat)
    g = g.reshape(b, ctx, emb_dim)
    h = _fc_in(g, W_in, b_in.reshape(1, -1))
    return _fc_out(h, W_out, b_out.reshape(1, -1))
